# baseline (speedup 1.0000x reference)
import jax
import jax.numpy as jnp
from jax.experimental import pallas as pl


_BLOCK_ROWS = 4800


def _copy2_kernel(xu_ref, xi_ref, ou_ref, oi_ref):
    ou_ref[...] = xu_ref[...]
    oi_ref[...] = xi_ref[...]


def kernel(x_user, x_item, edge_index_user_item, edge_index_item_user):
    del edge_index_user_item, edge_index_item_user
    n, d = x_user.shape
    block_rows = min(_BLOCK_ROWS, n)
    grid = ((n + block_rows - 1) // block_rows,)
    spec = pl.BlockSpec((block_rows, d), lambda i: (i, 0))
    out_u, out_i = pl.pallas_call(
        _copy2_kernel,
        grid=grid,
        in_specs=[spec, spec],
        out_specs=[spec, spec],
        out_shape=[
            jax.ShapeDtypeStruct(x_user.shape, x_user.dtype),
            jax.ShapeDtypeStruct(x_item.shape, x_item.dtype),
        ],
    )(x_user, x_item)
    return (out_u, out_i)


# speedup vs baseline: 1.0330x; 1.0330x over previous
import jax
import jax.numpy as jnp
from jax.experimental import pallas as pl


_BLOCK_ROWS = 4000


def _copy2_kernel(xu_ref, xi_ref, ou_ref, oi_ref):
    ou_ref[...] = xu_ref[...]
    oi_ref[...] = xi_ref[...]


def kernel(x_user, x_item, edge_index_user_item, edge_index_item_user):
    del edge_index_user_item, edge_index_item_user
    n, d = x_user.shape
    block_rows = min(_BLOCK_ROWS, n)
    grid = ((n + block_rows - 1) // block_rows,)
    spec = pl.BlockSpec((block_rows, d), lambda i: (i, 0))
    out_u, out_i = pl.pallas_call(
        _copy2_kernel,
        grid=grid,
        in_specs=[spec, spec],
        out_specs=[spec, spec],
        out_shape=[
            jax.ShapeDtypeStruct(x_user.shape, x_user.dtype),
            jax.ShapeDtypeStruct(x_item.shape, x_item.dtype),
        ],
    )(x_user, x_item)
    return (out_u, out_i)


# speedup vs baseline: 1.0342x; 1.0012x over previous
import jax
import jax.numpy as jnp
from jax.experimental import pallas as pl
from jax.experimental.pallas import tpu as pltpu


_BLOCK_ROWS = 4000


def _copy2_kernel(xu_ref, xi_ref, ou_ref, oi_ref):
    ou_ref[...] = xu_ref[...]
    oi_ref[...] = xi_ref[...]


def kernel(x_user, x_item, edge_index_user_item, edge_index_item_user):
    del edge_index_user_item, edge_index_item_user
    n, d = x_user.shape
    block_rows = min(_BLOCK_ROWS, n)
    grid = ((n + block_rows - 1) // block_rows,)
    spec = pl.BlockSpec((block_rows, d), lambda i: (i, 0))
    out_u, out_i = pl.pallas_call(
        _copy2_kernel,
        grid=grid,
        in_specs=[spec, spec],
        out_specs=[spec, spec],
        out_shape=[
            jax.ShapeDtypeStruct(x_user.shape, x_user.dtype),
            jax.ShapeDtypeStruct(x_item.shape, x_item.dtype),
        ],
        compiler_params=pltpu.CompilerParams(
            dimension_semantics=("parallel",),
        ),
    )(x_user, x_item)
    return (out_u, out_i)
